# Initial kernel scaffold; baseline (speedup 1.0000x reference)
import jax
import jax.numpy as jnp
from jax.experimental import pallas as pl


def kernel(target_bucket_emb, target_importance, context_bucket_emb, context_importance, target, context):
    raise NotImplementedError("write your pallas kernel here")


# speedup vs baseline: 2.7995x; 2.7995x over previous
import functools

import jax
import jax.numpy as jnp
from jax import lax
from jax.experimental import pallas as pl
from jax.experimental.pallas import tpu as pltpu
from jax.experimental.pallas import tpu_sc as plsc

_NUM_WORDS = 1000
_NUM_BUCKETS = 100000
_EMB = 64
_BATCH = 16384
_NUM_NEG = 4
_NOUT = 1 + _NUM_NEG
_HASH_A = (92821, 48271)
_HASH_B = (1013, 2029)

_WPAD = 1024
_NC, _NS = 2, 16
_NW = _NC * _NS
_BPW = _BATCH // _NW
_WPT = _WPAD // _NS
_NGRP = _BPW // 16


def _hash_ids(v):
    return (v * 1000003 + 12345) % _NUM_WORDS


def _sc_body(t_emb, t_imp, c_emb, c_imp, tgt, ctx, neg,
             out,
             tbl, we_t, rows, chunk, ids_v, neg_v, outv, idx_v, imp_v,
             wt_sp, wc_sp, sem):
    cid = lax.axis_index("c")
    sid = lax.axis_index("s")
    wid = cid * _NS + sid
    base = wid * _BPW
    wbase = sid * _WPT
    iota = lax.iota(jnp.int32, 16)

    for emb_hbm, imp_hbm, sp in ((t_emb, t_imp, wt_sp), (c_emb, c_imp, wc_sp)):
        for ch in range(8):
            pos = ch * 16 + iota
            w = wbase + (pos >> 1)
            even = (pos % 2) == 0
            a = jnp.where(even, jnp.full((16,), _HASH_A[0], jnp.int32),
                          jnp.full((16,), _HASH_A[1], jnp.int32))
            b = jnp.where(even, jnp.full((16,), _HASH_B[0], jnp.int32),
                          jnp.full((16,), _HASH_B[1], jnp.int32))
            idx_v[pl.ds(ch * 16, 16)] = (w * a + b) % _NUM_BUCKETS
        pltpu.async_copy(emb_hbm.at[idx_v], rows, sem).wait()
        pltpu.sync_copy(imp_hbm.at[pl.ds(wbase, _WPT), :], imp_v)

        def build_one(i, _):
            iv = jnp.full((16,), i, jnp.int32)
            z = jnp.zeros((16,), jnp.int32)
            f0 = plsc.load_gather(imp_v, [iv, z])
            f1 = plsc.load_gather(imp_v, [iv, z + 1])
            for sl in range(4):
                v = (f0 * rows[2 * i, pl.ds(sl * 16, 16)]
                     + f1 * rows[2 * i + 1, pl.ds(sl * 16, 16)])
                chunk[i, pl.ds(sl * 16, 16)] = v
            return 0

        lax.fori_loop(0, _WPT, build_one, 0)
        pltpu.sync_copy(chunk, sp.at[pl.ds(wbase, _WPT), :])

    plsc.subcore_barrier()

    pltpu.sync_copy(wt_sp, tbl)
    pltpu.sync_copy(tgt.at[pl.ds(base, _BPW)], ids_v)

    def tgt_group(g, _):
        t = _hash_ids(ids_v[pl.ds(g * 16, 16)])

        def ebody(e, _):
            ev = jnp.full((16,), e, jnp.int32)
            wev = plsc.load_gather(tbl, [t, ev])
            we_t[pl.ds(e * _BPW + g * 16, 16)] = wev
            return 0

        lax.fori_loop(0, _EMB, ebody, 0)
        return 0

    lax.fori_loop(0, _NGRP, tgt_group, 0)

    pltpu.sync_copy(wc_sp, tbl)
    pltpu.sync_copy(ctx.at[pl.ds(base, _BPW)], ids_v)
    pltpu.sync_copy(neg.at[pl.ds(base * _NUM_NEG, _BPW * _NUM_NEG)], neg_v)

    def ctx_group(g, _):
        rel = g * 16 + iota
        cids = [_hash_ids(ids_v[pl.ds(g * 16, 16)])]
        for j in range(_NUM_NEG):
            cids.append(plsc.load_gather(neg_v, [rel * _NUM_NEG + j]))

        def ebody(e, accs):
            ev = jnp.full((16,), e, jnp.int32)
            wev = we_t[pl.ds(e * _BPW + g * 16, 16)]
            return tuple(acc + plsc.load_gather(tbl, [c, ev]) * wev
                         for acc, c in zip(accs, cids))

        zero = jnp.zeros((16,), jnp.float32)
        accs = lax.fori_loop(0, _EMB, ebody, (zero,) * _NOUT)
        for j in range(_NOUT):
            plsc.store_scatter(outv, [rel * _NOUT + j], accs[j])
        return 0

    lax.fori_loop(0, _NGRP, ctx_group, 0)
    pltpu.sync_copy(outv, out.at[pl.ds(wid * _BPW * _NOUT, _BPW * _NOUT)])


_launch = functools.partial(
    pl.kernel,
    out_type=jax.ShapeDtypeStruct((_BATCH * _NOUT,), jnp.float32),
    mesh=plsc.VectorSubcoreMesh(core_axis_name="c", subcore_axis_name="s"),
    compiler_params=pltpu.CompilerParams(use_tc_tiling_on_sc=False,
                                         needs_layout_passes=False),
    scratch_types=[
        pltpu.VMEM((_WPAD, _EMB), jnp.float32),
        pltpu.VMEM((_BPW * _EMB,), jnp.float32),
        pltpu.VMEM((2 * _WPT, _EMB), jnp.float32),
        pltpu.VMEM((_WPT, _EMB), jnp.float32),
        pltpu.VMEM((_BPW,), jnp.int32),
        pltpu.VMEM((_BPW * _NUM_NEG,), jnp.int32),
        pltpu.VMEM((_BPW * _NOUT,), jnp.float32),
        pltpu.VMEM((2 * _WPT,), jnp.int32),
        pltpu.VMEM((_WPT, 2), jnp.float32),
        pltpu.VMEM_SHARED((_WPAD, _EMB), jnp.float32),
        pltpu.VMEM_SHARED((_WPAD, _EMB), jnp.float32),
        pltpu.SemaphoreType.DMA,
    ],
)(_sc_body)


def kernel(target_bucket_emb, target_importance, context_bucket_emb,
           context_importance, target, context):
    freqs = jnp.full((_NUM_WORDS,), 5.0, dtype=jnp.float32)
    dist = jnp.floor(jnp.power(freqs, 0.75) + 1.0)
    neg = jax.random.categorical(jax.random.key(1139), jnp.log(dist),
                                 shape=(_BATCH * _NUM_NEG,)).astype(jnp.int32)

    pad = ((0, _WPAD - _NUM_WORDS), (0, 0))
    t_imp = jnp.pad(target_importance, pad)
    c_imp = jnp.pad(context_importance, pad)
    tgt = target.reshape(-1).astype(jnp.int32)
    ctx = context.reshape(-1).astype(jnp.int32)

    out = _launch(target_bucket_emb, t_imp, context_bucket_emb, c_imp,
                  tgt, ctx, neg)
    return out.reshape(_BATCH, _NOUT)
